# baseline (speedup 1.0000x reference)
import functools
import math

import jax
import jax.numpy as jnp
from jax import lax
from jax.experimental import pallas as pl
from jax.experimental.pallas import tpu as pltpu

_B, _L, _D = 4096, 200, 64
_NTB = 64
_TWO_PI = 2.0 * math.pi


def _body(seq_ref, cur_ref, tt_ref, ss_ref, wp_ref, bp_ref, wf_ref, bf_ref,
          out_ref, *, tb):
    f32 = jnp.float32
    wf = wf_ref[...]
    wf1 = wf[:, 0:64]
    wf2 = wf[:, 64:128]
    wf3 = wf[:, 128:192]
    dn = (((1,), (1,)), ((), ()))
    tf = lax.dot_general(tt_ref[...], wf1, dn, preferred_element_type=f32)
    sf = lax.dot_general(ss_ref[...], wf2, dn, preferred_element_type=f32)
    m = lax.dot_general(wp_ref[...], wf3, (((0,), (1,)), ((), ())),
                        preferred_element_type=f32)
    m8 = jnp.concatenate([m, jnp.zeros((4, _D), f32)], axis=0)
    c = lax.dot_general(bp_ref[...], wf3, dn, preferred_element_type=f32)
    c = c + bf_ref[...]

    seq = seq_ref[...]
    cur = cur_ref[...]
    delta = jnp.maximum(cur - seq, 0)

    dm1 = jnp.maximum(delta.astype(f32), 1.0) / 60.0 + 1.0
    ebits = lax.bitcast_convert_type(dm1, jnp.int32)
    bidx = jnp.clip((ebits >> 23) - 127, 0, _NTB - 1)

    level = (delta > 1800).astype(jnp.int32) + (delta > 86400).astype(jnp.int32)

    n = tb * _L

    def minor(x, k):
        return lax.broadcast_in_dim(x, (tb, _L, k), (0, 1))

    io64 = lax.broadcasted_iota(jnp.int32, (tb, _L, _NTB), 2)
    oh_t = (io64 == minor(bidx, _NTB)).astype(f32).reshape(n, _NTB)
    io8 = lax.broadcasted_iota(jnp.int32, (tb, _L, 8), 2)
    oh_s = (io8 == minor(level, 8)).astype(f32).reshape(n, 8)

    seqb = minor(seq, 8)
    hour_b = lax.rem(seqb, 86400)
    a1 = hour_b.astype(f32) * f32(_TWO_PI / 86400.0)
    day = lax.rem(seqb.astype(f32) / 86400.0, 7.0)
    a2 = day * f32(_TWO_PI / 7.0)
    zero = jnp.zeros((), f32)
    per = (jnp.where(io8 == 0, jnp.sin(a1), zero)
           + jnp.where(io8 == 1, jnp.cos(a1), zero)
           + jnp.where(io8 == 2, jnp.sin(a2), zero)
           + jnp.where(io8 == 3, jnp.cos(a2), zero)).reshape(n, 8)

    acc = lax.dot_general(oh_t, tf, (((1,), (0,)), ((), ())),
                          preferred_element_type=f32)
    acc = acc + lax.dot_general(oh_s, sf, (((1,), (0,)), ((), ())),
                                preferred_element_type=f32)
    acc = acc + lax.dot_general(per, m8, (((1,), (0,)), ((), ())),
                                preferred_element_type=f32)
    out_ref[...] = acc + c


def kernel(seq_timestamps, current_timestamp, time_table, sess_table, Wp, bp, Wf, bf):
    tb = 128
    grid = _B // tb
    cur2 = current_timestamp.reshape(_B, 1)
    sess8 = jnp.concatenate(
        [sess_table, jnp.zeros((8 - sess_table.shape[0], _D), jnp.float32)], axis=0)
    bp2 = bp.reshape(1, _D)
    bf2 = bf.reshape(1, _D)

    full = lambda shape: pl.BlockSpec(shape, lambda i: (0, 0))
    out_flat = pl.pallas_call(
        functools.partial(_body, tb=tb),
        grid=(grid,),
        in_specs=[
            pl.BlockSpec((tb, _L), lambda i: (i, 0)),
            pl.BlockSpec((tb, 1), lambda i: (i, 0)),
            full((_NTB, _D)),
            full((8, _D)),
            full((_D, 4)),
            full((1, _D)),
            full((_D, 3 * _D)),
            full((1, _D)),
        ],
        out_specs=pl.BlockSpec((tb * _L, _D), lambda i: (i, 0)),
        out_shape=jax.ShapeDtypeStruct((_B * _L, _D), jnp.float32),
        compiler_params=pltpu.CompilerParams(
            dimension_semantics=("parallel",)),
    )(seq_timestamps, cur2, time_table, sess8, Wp, bp2, Wf, bf2)
    return out_flat.reshape(_B, _L, _D)


# speedup vs baseline: 6.2907x; 6.2907x over previous
import functools
import math

import jax
import jax.numpy as jnp
from jax import lax
from jax.experimental import pallas as pl
from jax.experimental.pallas import tpu as pltpu

_B, _L, _D = 4096, 200, 64
_NTB = 64
_TWO_PI = 2.0 * math.pi


def _body(seq_ref, cur_ref, tt_ref, ss_ref, wp_ref, bp_ref, wf_ref, bf_ref,
          out_ref, *, tb):
    f32 = jnp.float32
    wf = wf_ref[...]
    wf1 = wf[:, 0:64]
    wf2 = wf[:, 64:128]
    wf3 = wf[:, 128:192]
    dn = (((1,), (1,)), ((), ()))
    tf = lax.dot_general(tt_ref[...], wf1, dn, preferred_element_type=f32)
    sf = lax.dot_general(ss_ref[...], wf2, dn, preferred_element_type=f32)
    m = lax.dot_general(wp_ref[...], wf3, (((0,), (1,)), ((), ())),
                        preferred_element_type=f32)
    m8 = jnp.concatenate([m, jnp.zeros((4, _D), f32)], axis=0)
    c = lax.dot_general(bp_ref[...], wf3, dn, preferred_element_type=f32)
    c = c + bf_ref[...]

    seq = seq_ref[...]
    cur = cur_ref[...]
    delta = jnp.maximum(cur - seq, 0)

    dm1 = jnp.maximum(delta.astype(f32), 1.0) / 60.0 + 1.0
    ebits = lax.bitcast_convert_type(dm1, jnp.int32)
    bidx = jnp.clip((ebits >> 23) - 127, 0, _NTB - 1)

    level = (delta > 1800).astype(jnp.int32) + (delta > 86400).astype(jnp.int32)

    n = tb * _L

    def minor(x, k):
        return lax.broadcast_in_dim(x, (tb, _L, k), (0, 1))

    io64 = lax.broadcasted_iota(jnp.int32, (tb, _L, _NTB), 2)
    oh_t = (io64 == minor(bidx, _NTB)).astype(f32).reshape(n, _NTB)
    io8 = lax.broadcasted_iota(jnp.int32, (tb, _L, 8), 2)
    oh_s = (io8 == minor(level, 8)).astype(f32).reshape(n, 8)

    hour_i = lax.rem(seq, 86400)
    a1 = hour_i.astype(f32) * f32(_TWO_PI / 86400.0)
    day = lax.rem(seq.astype(f32) / 86400.0, 7.0)
    a2 = day * f32(_TWO_PI / 7.0)

    def pack2(x, y):
        xb = lax.bitcast_convert_type(x, jnp.int32)
        yb = lax.bitcast_convert_type(y, jnp.int32)
        xr = (xb + 0x8000) & jnp.int32(0xFFFF0000 - 0x100000000)
        return xr | ((yb + 0x8000) >> 16) & 0xFFFF

    p1 = pack2(jnp.sin(a1), jnp.cos(a1))
    p2 = pack2(jnp.sin(a2), jnp.cos(a2))
    p1b = minor(p1, 8)
    p2b = minor(p2, 8)

    def hi(p):
        return lax.bitcast_convert_type(p & jnp.int32(0xFFFF0000 - 0x100000000), f32)

    def lo(p):
        return lax.bitcast_convert_type(p << 16, f32)

    zero = jnp.zeros((), f32)
    per = (jnp.where(io8 == 0, hi(p1b), zero)
           + jnp.where(io8 == 1, lo(p1b), zero)
           + jnp.where(io8 == 2, hi(p2b), zero)
           + jnp.where(io8 == 3, lo(p2b), zero)).reshape(n, 8)

    acc = lax.dot_general(oh_t, tf, (((1,), (0,)), ((), ())),
                          preferred_element_type=f32)
    acc = acc + lax.dot_general(oh_s, sf, (((1,), (0,)), ((), ())),
                                preferred_element_type=f32)
    acc = acc + lax.dot_general(per, m8, (((1,), (0,)), ((), ())),
                                preferred_element_type=f32)
    out_ref[...] = acc + c


def kernel(seq_timestamps, current_timestamp, time_table, sess_table, Wp, bp, Wf, bf):
    tb = 128
    grid = _B // tb
    cur2 = current_timestamp.reshape(_B, 1)
    sess8 = jnp.concatenate(
        [sess_table, jnp.zeros((8 - sess_table.shape[0], _D), jnp.float32)], axis=0)
    bp2 = bp.reshape(1, _D)
    bf2 = bf.reshape(1, _D)

    full = lambda shape: pl.BlockSpec(shape, lambda i: (0, 0))
    out_flat = pl.pallas_call(
        functools.partial(_body, tb=tb),
        grid=(grid,),
        in_specs=[
            pl.BlockSpec((tb, _L), lambda i: (i, 0)),
            pl.BlockSpec((tb, 1), lambda i: (i, 0)),
            full((_NTB, _D)),
            full((8, _D)),
            full((_D, 4)),
            full((1, _D)),
            full((_D, 3 * _D)),
            full((1, _D)),
        ],
        out_specs=pl.BlockSpec((tb * _L, _D), lambda i: (i, 0)),
        out_shape=jax.ShapeDtypeStruct((_B * _L, _D), jnp.float32),
        compiler_params=pltpu.CompilerParams(
            dimension_semantics=("parallel",)),
    )(seq_timestamps, cur2, time_table, sess8, Wp, bp2, Wf, bf2)
    return out_flat.reshape(_B, _L, _D)
